# Initial kernel scaffold; baseline (speedup 1.0000x reference)
import jax
import jax.numpy as jnp
from jax.experimental import pallas as pl


def kernel(features_tensor, all_edge_index, all_subset, remains, all_mapping, inj_num, W1, b1, W2, b2, Wg, bg, noise):
    raise NotImplementedError("write your pallas kernel here")


# speedup vs baseline: 3.7320x; 3.7320x over previous
import functools

import jax
import jax.numpy as jnp
from jax import lax
from jax.experimental import pallas as pl
from jax.experimental.pallas import tpu as pltpu
from jax.experimental.pallas import tpu_sc as plsc

NC = 2
NS = 16
NW = NC * NS
D = 128
ROWS = 4096


def _leaky(x):
    return jnp.where(x >= 0, x, 0.01 * x)


def _sc_gather_deg(features, subset_w, dst_w, n_inj):
    rpw = ROWS // NW

    @functools.partial(
        pl.kernel,
        out_type=(
            jax.ShapeDtypeStruct((n_inj * ROWS, D), jnp.float32),
            jax.ShapeDtypeStruct((NW, n_inj, ROWS), jnp.float32),
        ),
        mesh=plsc.VectorSubcoreMesh(core_axis_name="c", subcore_axis_name="s"),
        compiler_params=pltpu.CompilerParams(needs_layout_passes=False),
        scratch_types=[
            pltpu.VMEM((rpw,), jnp.int32),
            pltpu.VMEM((rpw, D), jnp.float32),
            pltpu.VMEM((16, D), jnp.int32),
            pltpu.VMEM((ROWS,), jnp.float32),
            pltpu.SemaphoreType.DMA,
        ],
    )
    def k(feat_hbm, sub_hbm, dst_hbm,
          fs_out, deg_out, idx_v, rows_v, dst_v, deg_v, sem):
        cid = lax.axis_index("c")
        sid = lax.axis_index("s")
        wid = sid * NC + cid

        ones16 = jnp.full((16,), 1.0, jnp.float32)
        zeros16 = jnp.zeros((16,), jnp.float32)
        for inj in range(n_inj):
            pltpu.sync_copy(sub_hbm.at[inj, wid], idx_v)
            pltpu.async_copy(feat_hbm.at[idx_v], rows_v, sem).wait()
            pltpu.sync_copy(
                rows_v, fs_out.at[pl.ds(inj * ROWS + wid * rpw, rpw)])

            def zbody(t, carry):
                deg_v[pl.ds(t * 16, 16)] = zeros16
                return carry

            lax.fori_loop(0, ROWS // 16, zbody, 0)
            pltpu.sync_copy(dst_hbm.at[inj, wid], dst_v)

            def body(t, carry):
                j = t // 8
                kk = t % 8
                idx16 = dst_v[j, pl.ds(kk * 16, 16)]
                plsc.addupdate_scatter(deg_v, [idx16], ones16)
                return carry

            lax.fori_loop(0, 128, body, 0)
            pltpu.sync_copy(deg_v, deg_out.at[wid, inj])

    return k(features, subset_w, dst_w)


def _sc_edge_agg(xw, src_w, dst_w, zeros_rows, n_inj):
    nchunk = src_w.shape[2]

    @functools.partial(
        pl.kernel,
        out_type=jax.ShapeDtypeStruct((NC, n_inj, ROWS, D), jnp.float32),
        mesh=plsc.VectorSubcoreMesh(core_axis_name="c", subcore_axis_name="s"),
        scratch_types=[
            pltpu.VMEM((nchunk, D), jnp.int32),
            pltpu.VMEM((nchunk, D), jnp.int32),
            pltpu.VMEM((2, D, D), jnp.float32),
            pltpu.VMEM_SHARED((ROWS, D), jnp.float32),
            pltpu.SemaphoreType.DMA,
            pltpu.SemaphoreType.DMA,
        ],
    )
    def k(xw_hbm, src_hbm, dst_hbm, z_hbm, agg_out,
          src_v, dst_v, rows_v, accsh, sem0, sem1):
        cid = lax.axis_index("c")
        sid = lax.axis_index("s")
        wid = sid * NC + cid
        sems = [sem0, sem1]
        rpt = ROWS // NS

        for inj in range(n_inj):
            pltpu.sync_copy(z_hbm.at[pl.ds(0, rpt)],
                            accsh.at[pl.ds(sid * rpt, rpt)])
            plsc.subcore_barrier()
            pltpu.sync_copy(src_hbm.at[inj, wid], src_v)
            pltpu.sync_copy(dst_hbm.at[inj, wid], dst_v)

            handles = [None, None]
            handles[0] = pltpu.async_copy(
                xw_hbm.at[src_v.at[0]], rows_v.at[0], sems[0])
            for j in range(nchunk):
                b = j % 2
                handles[b].wait()
                if j + 1 < nchunk:
                    nb = (j + 1) % 2
                    handles[nb] = pltpu.async_copy(
                        xw_hbm.at[src_v.at[j + 1]], rows_v.at[nb], sems[nb])
                pltpu.sync_copy(rows_v.at[b], accsh.at[dst_v.at[j]], add=True)

            plsc.subcore_barrier()
            pltpu.sync_copy(accsh.at[pl.ds(sid * rpt, rpt)],
                            agg_out.at[cid, inj, pl.ds(sid * rpt, rpt)])

    return k(xw, src_w, dst_w, zeros_rows)


def _tc_b1(fs_all, deg2, W1):
    M = fs_all.shape[0]
    TM = 512

    def body(fs, deg, w, xw_out, dinv_out):
        d = jnp.sum(deg[...], axis=0) + 1.0
        dinv = lax.rsqrt(d)
        xw = jnp.dot(fs[...], w[...], preferred_element_type=jnp.float32)
        xw_out[...] = xw * dinv
        dinv_out[...] = dinv

    return pl.pallas_call(
        body,
        grid=(M // TM,),
        in_specs=[
            pl.BlockSpec((TM, D), lambda m: (m, 0)),
            pl.BlockSpec((NW, TM, 1), lambda m: (0, m, 0)),
            pl.BlockSpec((D, D), lambda m: (0, 0)),
        ],
        out_specs=[
            pl.BlockSpec((TM, D), lambda m: (m, 0)),
            pl.BlockSpec((TM, 1), lambda m: (m, 0)),
        ],
        out_shape=[
            jax.ShapeDtypeStruct((M, D), jnp.float32),
            jax.ShapeDtypeStruct((M, 1), jnp.float32),
        ],
    )(fs_all, deg2, W1)


def _tc_b2(agg2, xw1, dinv, W2, b1):
    M = xw1.shape[0]
    TM = 512

    def body(ag, xw, dv, w, b, out):
        tot = ag[0] + ag[1] + xw[...]
        h = _leaky(tot * dv[...] + b[...])
        out[...] = jnp.dot(h, w[...], preferred_element_type=jnp.float32) * dv[...]

    return pl.pallas_call(
        body,
        grid=(M // TM,),
        in_specs=[
            pl.BlockSpec((2, TM, D), lambda m: (0, m, 0)),
            pl.BlockSpec((TM, D), lambda m: (m, 0)),
            pl.BlockSpec((TM, 1), lambda m: (m, 0)),
            pl.BlockSpec((D, D), lambda m: (0, 0)),
            pl.BlockSpec((1, D), lambda m: (0, 0)),
        ],
        out_specs=pl.BlockSpec((TM, D), lambda m: (m, 0)),
        out_shape=jax.ShapeDtypeStruct((M, D), jnp.float32),
    )(agg2, xw1, dinv, W2, b1)


def _tc_b3(agg2, xw2, dinv, fs_all, b2, remains, mapping, n_inj, n_nodes):
    n_rem = remains.shape[2]

    def body(ag, xw, dv, fs, b, tn, mp, hid_out, ru_out, h2_s):
        tot = ag[0, 0] + ag[1, 0] + xw[0]
        h2 = _leaky(tot * dv[0] + b[...])
        h2_s[...] = h2
        hm = jnp.sum(h2[:n_nodes, :], axis=0) * (1.0 / n_nodes)
        im = mp[0, 0, 0]
        inh = h2_s[pl.ds(im, 1), :]

        def rbody(j, carry):
            sh, sf = carry
            r = tn[0, 0, j]
            sh = sh + h2_s[pl.ds(r, 1), :]
            sf = sf + fs[0, pl.ds(r, 1), :]
            return (sh, sf)

        z = jnp.zeros((1, D), jnp.float32)
        sh, sf = lax.fori_loop(0, n_rem, rbody, (z, z))
        hid_out[0, 0, pl.ds(0, D)] = hm.reshape(1, D)[0]
        hid_out[0, 0, pl.ds(D, D)] = inh[0]
        hid_out[0, 0, pl.ds(2 * D, D)] = (sh * (1.0 / n_rem))[0]
        ru_out[0, 0, :] = (sf * (1.0 / n_rem))[0]

    return pl.pallas_call(
        body,
        grid=(n_inj,),
        in_specs=[
            pl.BlockSpec((2, 1, ROWS, D), lambda i: (0, i, 0, 0)),
            pl.BlockSpec((1, ROWS, D), lambda i: (i, 0, 0)),
            pl.BlockSpec((1, ROWS, 1), lambda i: (i, 0, 0)),
            pl.BlockSpec((1, ROWS, D), lambda i: (i, 0, 0)),
            pl.BlockSpec((1, D), lambda i: (0, 0)),
            pl.BlockSpec((1, 1, n_rem), lambda i: (i, 0, 0),
                         memory_space=pltpu.SMEM),
            pl.BlockSpec((1, 1, 1), lambda i: (i, 0, 0),
                         memory_space=pltpu.SMEM),
        ],
        out_specs=[
            pl.BlockSpec((1, 1, 3 * D), lambda i: (i, 0, 0)),
            pl.BlockSpec((1, 1, D), lambda i: (i, 0, 0)),
        ],
        out_shape=[
            jax.ShapeDtypeStruct((n_inj, 1, 3 * D), jnp.float32),
            jax.ShapeDtypeStruct((n_inj, 1, D), jnp.float32),
        ],
        scratch_shapes=[pltpu.VMEM((ROWS, D), jnp.float32)],
    )(agg2, xw2, dinv, fs_all, b2, remains, mapping)


def _tc_head(hidden, Wg, bg, noise, ru):
    n_inj = hidden.shape[0]

    def body(hid, wg, b, nz, r, feat_o, mu_o, sig_o, hom_o):
        fd = _leaky(jnp.dot(hid[...], wg[...],
                            preferred_element_type=jnp.float32) + b[...])
        mu = fd[:, :D]
        sigma = jnp.abs(fd[:, D:]) + 1e-9
        feat = jnp.clip(mu + sigma * nz[...], -1.0, 1.0)
        ru_ = r[...]
        num = jnp.sum(ru_ * feat, axis=1, keepdims=True)
        den = jnp.maximum(
            jnp.sqrt(jnp.sum(ru_ * ru_, axis=1, keepdims=True)) *
            jnp.sqrt(jnp.sum(feat * feat, axis=1, keepdims=True)), 1e-8)
        feat_o[...] = feat
        mu_o[...] = mu
        sig_o[...] = sigma
        hom_o[...] = (jnp.sum(num / den) * (1.0 / n_inj)).reshape(1, 1)

    return pl.pallas_call(
        body,
        in_specs=[
            pl.BlockSpec(hidden.shape, lambda: (0, 0)),
            pl.BlockSpec(Wg.shape, lambda: (0, 0)),
            pl.BlockSpec((1, 2 * D), lambda: (0, 0)),
            pl.BlockSpec(noise.shape, lambda: (0, 0)),
            pl.BlockSpec(ru.shape, lambda: (0, 0)),
        ],
        out_specs=[
            pl.BlockSpec((n_inj, D), lambda: (0, 0)),
            pl.BlockSpec((n_inj, D), lambda: (0, 0)),
            pl.BlockSpec((n_inj, D), lambda: (0, 0)),
            pl.BlockSpec((1, 1), lambda: (0, 0)),
        ],
        out_shape=[
            jax.ShapeDtypeStruct((n_inj, D), jnp.float32),
            jax.ShapeDtypeStruct((n_inj, D), jnp.float32),
            jax.ShapeDtypeStruct((n_inj, D), jnp.float32),
            jax.ShapeDtypeStruct((1, 1), jnp.float32),
        ],
    )(hidden, Wg, bg, noise, ru)


@jax.jit
def _run(features_tensor, all_edge_index, all_subset, remains, all_mapping,
         W1, b1, W2, b2, Wg, bg, noise):
    n_inj, S = all_subset.shape
    E = all_edge_index.shape[2]
    epw = -(-E // NW)
    epw = ((epw + D - 1) // D) * D
    EP = epw * NW
    nchunk = epw // D

    src = all_edge_index[:, 0, :].astype(jnp.int32)
    dst = all_edge_index[:, 1, :].astype(jnp.int32)
    inj_off = (jnp.arange(n_inj, dtype=jnp.int32) * ROWS)[:, None]
    src_g = jnp.concatenate(
        [src + inj_off, jnp.zeros((n_inj, EP - E), jnp.int32)], axis=1)
    dst_p = jnp.concatenate(
        [dst, jnp.full((n_inj, EP - E), ROWS - 1, jnp.int32)], axis=1)
    src_w = src_g.reshape(n_inj, NW, nchunk, D)
    dst_w = dst_p.reshape(n_inj, NW, nchunk, D)
    sub_p = jnp.concatenate(
        [all_subset.astype(jnp.int32),
         jnp.zeros((n_inj, ROWS - S), jnp.int32)], axis=1)
    sub_w = sub_p.reshape(n_inj, NW, ROWS // NW)
    zeros_rows = jnp.zeros((ROWS // NS, D), jnp.float32)
    remains = remains.astype(jnp.int32)
    mapping = all_mapping.astype(jnp.int32)

    fs_all, deg_part = _sc_gather_deg(features_tensor, sub_w, dst_w, n_inj)
    deg2 = deg_part.reshape(NW, n_inj * ROWS, 1)

    xw1, dinv = _tc_b1(fs_all, deg2, W1)

    agg1 = _sc_edge_agg(xw1, src_w, dst_w, zeros_rows, n_inj)
    xw2 = _tc_b2(agg1.reshape(NC, n_inj * ROWS, D), xw1, dinv, W2,
                 b1.reshape(1, D))

    agg2 = _sc_edge_agg(xw2, src_w, dst_w, zeros_rows, n_inj)

    hidden, ru = _tc_b3(
        agg2,
        xw2.reshape(n_inj, ROWS, D),
        dinv.reshape(n_inj, ROWS, 1),
        fs_all.reshape(n_inj, ROWS, D),
        b2.reshape(1, D), remains.reshape(n_inj, 1, -1),
        mapping.reshape(n_inj, 1, 1), n_inj, S)

    feat, mu, sigma, hom = _tc_head(
        hidden.reshape(n_inj, 3 * D), Wg, bg.reshape(1, 2 * D), noise,
        ru.reshape(n_inj, D))
    return feat, mu, sigma, hom[0, 0]


def kernel(features_tensor, all_edge_index, all_subset, remains, all_mapping,
           inj_num, W1, b1, W2, b2, Wg, bg, noise):
    return _run(features_tensor, all_edge_index, all_subset, remains,
                all_mapping, W1, b1, W2, b2, Wg, bg, noise)
